# baseline (speedup 1.0000x reference)
import jax
import jax.numpy as jnp
from jax import lax
from jax.experimental import pallas as pl
from jax.experimental.pallas import tpu as pltpu
from jax.experimental.pallas import tpu_sc as plsc

FEAT = 1000000
B = 4096
D = 32
NTILES = 16
CHUNK = B // NTILES
GROUPS = CHUNK // 16

LN2 = 0.6931471805599453
LN100 = 4.605170185988091
X_MAX_POW = 0.75


def _ln(x):
    bits = plsc.bitcast(x, jnp.int32)
    e = lax.shift_right_logical(bits, 23) & 0xFF
    e = e - 127
    m_bits = (bits & 0x007FFFFF) | 0x3F800000
    m = plsc.bitcast(m_bits, jnp.float32)
    big = m >= 1.4142135623730951
    m = jnp.where(big, m * 0.5, m)
    e = e + jnp.where(big, 1, 0)
    s = (m - 1.0) / (m + 1.0)
    s2 = s * s
    lnm = 2.0 * s * (1.0 + s2 * (1.0 / 3.0 + s2 * (0.2 + s2 * (1.0 / 7.0))))
    return LN2 * e.astype(jnp.float32) + lnm


def _glove_kernel(v_hbm, w_hbm, vb_hbm, wb_hbm, c_hbm, i1_hbm, i2_hbm, out_hbm,
                  i1_v, i2_v, fidx1, fidx2, vt, wt_t, vb_v, wb_v, c_v,
                  stage, shared, red_v, out_stage,
                  sem_v, sem_w, sem_vb, sem_wb):
    sid = lax.axis_index("s")
    base = sid * CHUNK

    pltpu.sync_copy(i1_hbm.at[pl.ds(base, CHUNK)], i1_v)
    pltpu.sync_copy(i2_hbm.at[pl.ds(base, CHUNK)], i2_v)
    pltpu.sync_copy(c_hbm.at[pl.ds(base, CHUNK)], c_v)

    cp_vb = pltpu.async_copy(vb_hbm.at[i1_v], vb_v, sem_vb)
    cp_wb = pltpu.async_copy(wb_hbm.at[i2_v], wb_v, sem_wb)

    for g in range(GROUPS):
        gbase = g * 16
        idx1 = i1_v[pl.ds(gbase, 16)]
        idx2 = i2_v[pl.ds(gbase, 16)]
        for d in range(D):
            fidx1[pl.ds(d * CHUNK + gbase, 16)] = idx1 + d * FEAT
            fidx2[pl.ds(d * CHUNK + gbase, 16)] = idx2 + d * FEAT

    cp_v = pltpu.async_copy(v_hbm.at[fidx1], vt, sem_v)
    cp_w = pltpu.async_copy(w_hbm.at[fidx2], wt_t, sem_w)
    cp_v.wait()
    cp_w.wait()
    cp_vb.wait()
    cp_wb.wait()

    zero = jnp.zeros((16,), jnp.float32)
    s1 = zero
    s2 = zero
    s3 = zero
    t1 = zero
    t2 = zero
    for g in range(GROUPS):
        gbase = g * 16
        dotv = zero
        for d in range(D):
            off = d * CHUNK + gbase
            dotv = dotv + vt[pl.ds(off, 16)] * wt_t[pl.ds(off, 16)]
        c = c_v[pl.ds(gbase, 16)]
        lnc = _ln(c)
        a = dotv + lnc
        wt = jnp.minimum(jnp.exp(X_MAX_POW * (lnc - LN100)), 1.0)
        s1 = s1 + wt * a * a
        s2 = s2 + wt * a
        s3 = s3 + wt
        bb = vb_v[pl.ds(gbase, 16)] + wb_v[pl.ds(gbase, 16)]
        t1 = t1 + bb
        t2 = t2 + bb * bb

    stage[pl.ds(0, 16)] = s1
    stage[pl.ds(16, 16)] = s2
    stage[pl.ds(32, 16)] = s3
    stage[pl.ds(48, 16)] = t1
    stage[pl.ds(64, 16)] = t2
    pltpu.sync_copy(stage, shared.at[sid])
    plsc.subcore_barrier()

    @pl.when(sid == 0)
    def _():
        pltpu.sync_copy(shared, red_v)
        a1 = zero
        a2 = zero
        a3 = zero
        a4 = zero
        a5 = zero
        for t in range(NTILES):
            a1 = a1 + red_v[t, pl.ds(0, 16)]
            a2 = a2 + red_v[t, pl.ds(16, 16)]
            a3 = a3 + red_v[t, pl.ds(32, 16)]
            a4 = a4 + red_v[t, pl.ds(48, 16)]
            a5 = a5 + red_v[t, pl.ds(64, 16)]
        S1 = jnp.sum(a1)
        S2 = jnp.sum(a2)
        S3 = jnp.sum(a3)
        T1 = jnp.sum(a4)
        T2 = jnp.sum(a5)
        final = float(B) * S1 + 2.0 * S2 * T1 + S3 * T2
        out_stage[...] = jnp.full((16,), final, jnp.float32)
        pltpu.sync_copy(out_stage, out_hbm)


def kernel(v_table, w_table, v_bias_table, w_bias_table, cooccur,
           feature_idx1, feature_idx2):
    mesh = plsc.VectorSubcoreMesh(
        core_axis_name="c", subcore_axis_name="s", num_cores=1)
    run = pl.kernel(
        _glove_kernel,
        out_type=jax.ShapeDtypeStruct((16,), jnp.float32),
        mesh=mesh,
        compiler_params=pltpu.CompilerParams(
            needs_layout_passes=False, use_tc_tiling_on_sc=False),
        scratch_types=[
            pltpu.VMEM((CHUNK,), jnp.int32),
            pltpu.VMEM((CHUNK,), jnp.int32),
            pltpu.VMEM((CHUNK * D,), jnp.int32),
            pltpu.VMEM((CHUNK * D,), jnp.int32),
            pltpu.VMEM((CHUNK * D,), jnp.float32),
            pltpu.VMEM((CHUNK * D,), jnp.float32),
            pltpu.VMEM((CHUNK,), jnp.float32),
            pltpu.VMEM((CHUNK,), jnp.float32),
            pltpu.VMEM((CHUNK,), jnp.float32),
            pltpu.VMEM((80,), jnp.float32),
            pltpu.VMEM_SHARED((NTILES, 80), jnp.float32),
            pltpu.VMEM((NTILES, 80), jnp.float32),
            pltpu.VMEM((16,), jnp.float32),
            pltpu.SemaphoreType.DMA,
            pltpu.SemaphoreType.DMA,
            pltpu.SemaphoreType.DMA,
            pltpu.SemaphoreType.DMA,
        ],
    )
    out = run(v_table.T.reshape(-1), w_table.T.reshape(-1),
              v_bias_table.reshape(-1), w_bias_table.reshape(-1),
              cooccur, feature_idx1.astype(jnp.int32),
              feature_idx2.astype(jnp.int32))
    return out[0]


# speedup vs baseline: 24.7263x; 24.7263x over previous
import jax
import jax.numpy as jnp
from jax import lax
from jax.experimental import pallas as pl
from jax.experimental.pallas import tpu as pltpu
from jax.experimental.pallas import tpu_sc as plsc

FEAT = 1000000
B = 4096
D = 32
NTILES = 16
CHUNK = B // NTILES
GROUPS = CHUNK // 16
GRAN = 128
NRING = 4

LN2 = 0.6931471805599453
LN100 = 4.605170185988091
X_MAX_POW = 0.75


def _ln(x):
    bits = plsc.bitcast(x, jnp.int32)
    e = lax.shift_right_logical(bits, 23) & 0xFF
    e = e - 127
    m_bits = (bits & 0x007FFFFF) | 0x3F800000
    m = plsc.bitcast(m_bits, jnp.float32)
    big = m >= 1.4142135623730951
    m = jnp.where(big, m * 0.5, m)
    e = e + jnp.where(big, 1, 0)
    s = (m - 1.0) / (m + 1.0)
    s2 = s * s
    lnm = 2.0 * s * (1.0 + s2 * (1.0 / 3.0 + s2 * (0.2 + s2 * (1.0 / 7.0))))
    return LN2 * e.astype(jnp.float32) + lnm


def _glove_kernel(v_hbm, w_hbm, vb_hbm, wb_hbm, c_hbm, i1_hbm, i2_hbm, out_hbm,
                  i1_v, i2_v, vbufs, wbufs, dots_ref,
                  vb_v, wb_v, c_v, stage, shared, red_v, out_stage,
                  sems_v, sems_w, sem_vb, sem_wb):
    sid = lax.axis_index("s")
    base = sid * CHUNK

    pltpu.sync_copy(i1_hbm.at[pl.ds(base, CHUNK)], i1_v)
    pltpu.sync_copy(i2_hbm.at[pl.ds(base, CHUNK)], i2_v)
    pltpu.sync_copy(c_hbm.at[pl.ds(base, CHUNK)], c_v)

    cp_vb = pltpu.async_copy(vb_hbm.at[i1_v], vb_v, sem_vb)
    cp_wb = pltpu.async_copy(wb_hbm.at[i2_v], wb_v, sem_wb)

    def _fire_ids(i, j, rb):
        off = pl.multiple_of(lax.shift_right_logical(i, 7) * 128, 128)
        pltpu.async_copy(v_hbm.at[:, pl.ds(off, GRAN)], vbufs[rb],
                         sems_v[rb])
        offj = pl.multiple_of(lax.shift_right_logical(j, 7) * 128, 128)
        pltpu.async_copy(w_hbm.at[:, pl.ds(offj, GRAN)], wbufs[rb],
                         sems_w[rb])

    iv_prime1 = i1_v[pl.ds(0, 16)]
    iv_prime2 = i2_v[pl.ds(0, 16)]
    for r in range(NRING):
        _fire_ids(iv_prime1[r], iv_prime2[r], r)

    iota = lax.broadcasted_iota(jnp.int32, (16,), 0)
    zero = jnp.zeros((16,), jnp.float32)

    def _main_body(g, carry):
        s1, s2, s3, t1, t2 = carry
        gb = pl.multiple_of(g * 16, 16)
        gb_next = pl.multiple_of(
            jnp.minimum(gb + 16, (GROUPS - 1) * 16), 16)
        iv1 = i1_v[pl.ds(gb, 16)]
        iv2 = i2_v[pl.ds(gb, 16)]
        ivn1 = i1_v[pl.ds(gb_next, 16)]
        ivn2 = i2_v[pl.ds(gb_next, 16)]
        dotg = zero
        for j in range(16):
            rb = j % NRING
            pltpu.make_async_copy(v_hbm.at[:, pl.ds(0, GRAN)], vbufs[rb],
                                  sems_v[rb]).wait()
            pltpu.make_async_copy(w_hbm.at[:, pl.ds(0, GRAN)], wbufs[rb],
                                  sems_w[rb]).wait()
            li = jnp.full((16,), iv1[j] & (GRAN - 1), jnp.int32)
            lj = jnp.full((16,), iv2[j] & (GRAN - 1), jnp.int32)
            v_lo = plsc.load_gather(vbufs[rb], [iota, li])
            v_hi = plsc.load_gather(vbufs[rb], [iota + 16, li])
            w_lo = plsc.load_gather(wbufs[rb], [iota, lj])
            w_hi = plsc.load_gather(wbufs[rb], [iota + 16, lj])
            dot_b = jnp.sum(v_lo * w_lo + v_hi * w_hi)
            dotg = jnp.where(iota == j, dot_b, dotg)
            if j < 16 - NRING:
                _fire_ids(iv1[j + NRING], iv2[j + NRING], rb)
            else:
                _fire_ids(ivn1[j - (16 - NRING)], ivn2[j - (16 - NRING)], rb)
        c = c_v[pl.ds(pl.multiple_of(gb, 16), 16)]
        lnc = _ln(c)
        a = dotg + lnc
        wt = jnp.minimum(jnp.exp(X_MAX_POW * (lnc - LN100)), 1.0)
        bb = (vb_v[pl.ds(pl.multiple_of(gb, 16), 16)]
              + wb_v[pl.ds(pl.multiple_of(gb, 16), 16)])
        return (s1 + wt * a * a, s2 + wt * a, s3 + wt,
                t1 + bb, t2 + bb * bb)

    cp_vb.wait()
    cp_wb.wait()
    s1, s2, s3, t1, t2 = lax.fori_loop(
        0, GROUPS, _main_body, (zero, zero, zero, zero, zero))

    for r in range(NRING):
        pltpu.make_async_copy(v_hbm.at[:, pl.ds(0, GRAN)], vbufs[r],
                              sems_v[r]).wait()
        pltpu.make_async_copy(w_hbm.at[:, pl.ds(0, GRAN)], wbufs[r],
                              sems_w[r]).wait()

    stage[pl.ds(0, 16)] = s1
    stage[pl.ds(16, 16)] = s2
    stage[pl.ds(32, 16)] = s3
    stage[pl.ds(48, 16)] = t1
    stage[pl.ds(64, 16)] = t2
    pltpu.sync_copy(stage, shared.at[pl.ds(sid * 80, 80)])
    plsc.subcore_barrier()

    @pl.when(sid == 0)
    def _():
        pltpu.sync_copy(shared, red_v)
        a1 = zero
        a2 = zero
        a3 = zero
        a4 = zero
        a5 = zero
        for t in range(NTILES):
            a1 = a1 + red_v[pl.ds(t * 80 + 0, 16)]
            a2 = a2 + red_v[pl.ds(t * 80 + 16, 16)]
            a3 = a3 + red_v[pl.ds(t * 80 + 32, 16)]
            a4 = a4 + red_v[pl.ds(t * 80 + 48, 16)]
            a5 = a5 + red_v[pl.ds(t * 80 + 64, 16)]
        S1 = jnp.sum(a1)
        S2 = jnp.sum(a2)
        S3 = jnp.sum(a3)
        T1 = jnp.sum(a4)
        T2 = jnp.sum(a5)
        final = float(B) * S1 + 2.0 * S2 * T1 + S3 * T2
        out_stage[...] = jnp.full((16,), final, jnp.float32)
        pltpu.sync_copy(out_stage, out_hbm)


def kernel(v_table, w_table, v_bias_table, w_bias_table, cooccur,
           feature_idx1, feature_idx2):
    mesh = plsc.VectorSubcoreMesh(
        core_axis_name="c", subcore_axis_name="s", num_cores=1)
    run = pl.kernel(
        _glove_kernel,
        out_type=jax.ShapeDtypeStruct((16,), jnp.float32),
        mesh=mesh,
        compiler_params=pltpu.CompilerParams(
            needs_layout_passes=False,
            use_tc_tiling_on_sc=True,
        ),
        scratch_types=[
            pltpu.VMEM((CHUNK,), jnp.int32),
            pltpu.VMEM((CHUNK,), jnp.int32),
            [pltpu.VMEM((D, GRAN), jnp.float32) for _ in range(NRING)],
            [pltpu.VMEM((D, GRAN), jnp.float32) for _ in range(NRING)],
            pltpu.VMEM((16,), jnp.float32),
            pltpu.VMEM((CHUNK,), jnp.float32),
            pltpu.VMEM((CHUNK,), jnp.float32),
            pltpu.VMEM((CHUNK,), jnp.float32),
            pltpu.VMEM((80,), jnp.float32),
            pltpu.VMEM_SHARED((NTILES * 80,), jnp.float32),
            pltpu.VMEM((NTILES * 80,), jnp.float32),
            pltpu.VMEM((16,), jnp.float32),
            [pltpu.SemaphoreType.DMA for _ in range(NRING)],
            [pltpu.SemaphoreType.DMA for _ in range(NRING)],
            pltpu.SemaphoreType.DMA,
            pltpu.SemaphoreType.DMA,
        ],
    )
    out = run(v_table.T, w_table.T,
              v_bias_table.reshape(-1), w_bias_table.reshape(-1),
              cooccur, feature_idx1.astype(jnp.int32),
              feature_idx2.astype(jnp.int32))
    return out[0]


# speedup vs baseline: 30.5521x; 1.2356x over previous
import jax
import jax.numpy as jnp
from jax import lax
from jax.experimental import pallas as pl
from jax.experimental.pallas import tpu as pltpu
from jax.experimental.pallas import tpu_sc as plsc

FEAT = 1000000
B = 4096
D = 32
NCORES = 2
NTILES = 16
NWORK = NCORES * NTILES
CHUNK = B // NWORK
GROUPS = CHUNK // 16
GRAN = 128
NRING = 4

LN2 = 0.6931471805599453
LN100 = 4.605170185988091
X_MAX_POW = 0.75


def _ln(x):
    bits = plsc.bitcast(x, jnp.int32)
    e = lax.shift_right_logical(bits, 23) & 0xFF
    e = e - 127
    m_bits = (bits & 0x007FFFFF) | 0x3F800000
    m = plsc.bitcast(m_bits, jnp.float32)
    big = m >= 1.4142135623730951
    m = jnp.where(big, m * 0.5, m)
    e = e + jnp.where(big, 1, 0)
    s = (m - 1.0) / (m + 1.0)
    s2 = s * s
    lnm = 2.0 * s * (1.0 + s2 * (1.0 / 3.0 + s2 * (0.2 + s2 * (1.0 / 7.0))))
    return LN2 * e.astype(jnp.float32) + lnm


def _glove_kernel(v_hbm, w_hbm, vb_hbm, wb_hbm, c_hbm, i1_hbm, i2_hbm, out_hbm,
                  i1_v, i2_v, vbufs, wbufs, dots_ref,
                  vb_v, wb_v, c_v, stage, shared, red_v, out_stage,
                  sems_v, sems_w, sem_vb, sem_wb):
    sid = lax.axis_index("s")
    cid = lax.axis_index("c")
    wid = cid * NTILES + sid
    base = wid * CHUNK

    pltpu.sync_copy(i1_hbm.at[pl.ds(base, CHUNK)], i1_v)
    pltpu.sync_copy(i2_hbm.at[pl.ds(base, CHUNK)], i2_v)
    pltpu.sync_copy(c_hbm.at[pl.ds(base, CHUNK)], c_v)

    cp_vb = pltpu.async_copy(vb_hbm.at[i1_v], vb_v, sem_vb)
    cp_wb = pltpu.async_copy(wb_hbm.at[i2_v], wb_v, sem_wb)

    def _fire_ids(i, j, rb):
        off = pl.multiple_of(lax.shift_right_logical(i, 7) * 128, 128)
        pltpu.async_copy(v_hbm.at[:, pl.ds(off, GRAN)], vbufs[rb],
                         sems_v[rb])
        offj = pl.multiple_of(lax.shift_right_logical(j, 7) * 128, 128)
        pltpu.async_copy(w_hbm.at[:, pl.ds(offj, GRAN)], wbufs[rb],
                         sems_w[rb])

    iv_prime1 = i1_v[pl.ds(0, 16)]
    iv_prime2 = i2_v[pl.ds(0, 16)]
    for r in range(NRING):
        _fire_ids(iv_prime1[r], iv_prime2[r], r)

    iota = lax.broadcasted_iota(jnp.int32, (16,), 0)
    zero = jnp.zeros((16,), jnp.float32)

    def _main_body(g, carry):
        s1, s2, s3, t1, t2 = carry
        gb = pl.multiple_of(g * 16, 16)
        gb_next = pl.multiple_of(
            jnp.minimum(gb + 16, (GROUPS - 1) * 16), 16)
        iv1 = i1_v[pl.ds(gb, 16)]
        iv2 = i2_v[pl.ds(gb, 16)]
        ivn1 = i1_v[pl.ds(gb_next, 16)]
        ivn2 = i2_v[pl.ds(gb_next, 16)]
        dotg = zero
        for j in range(16):
            rb = j % NRING
            pltpu.make_async_copy(v_hbm.at[:, pl.ds(0, GRAN)], vbufs[rb],
                                  sems_v[rb]).wait()
            pltpu.make_async_copy(w_hbm.at[:, pl.ds(0, GRAN)], wbufs[rb],
                                  sems_w[rb]).wait()
            li = jnp.full((16,), iv1[j] & (GRAN - 1), jnp.int32)
            lj = jnp.full((16,), iv2[j] & (GRAN - 1), jnp.int32)
            v_lo = plsc.load_gather(vbufs[rb], [iota, li])
            v_hi = plsc.load_gather(vbufs[rb], [iota + 16, li])
            w_lo = plsc.load_gather(wbufs[rb], [iota, lj])
            w_hi = plsc.load_gather(wbufs[rb], [iota + 16, lj])
            dot_b = jnp.sum(v_lo * w_lo + v_hi * w_hi)
            dotg = jnp.where(iota == j, dot_b, dotg)
            if j < 16 - NRING:
                _fire_ids(iv1[j + NRING], iv2[j + NRING], rb)
            else:
                _fire_ids(ivn1[j - (16 - NRING)], ivn2[j - (16 - NRING)], rb)
        c = c_v[pl.ds(pl.multiple_of(gb, 16), 16)]
        lnc = _ln(c)
        a = dotg + lnc
        wt = jnp.minimum(jnp.exp(X_MAX_POW * (lnc - LN100)), 1.0)
        bb = (vb_v[pl.ds(pl.multiple_of(gb, 16), 16)]
              + wb_v[pl.ds(pl.multiple_of(gb, 16), 16)])
        return (s1 + wt * a * a, s2 + wt * a, s3 + wt,
                t1 + bb, t2 + bb * bb)

    cp_vb.wait()
    cp_wb.wait()
    s1, s2, s3, t1, t2 = lax.fori_loop(
        0, GROUPS, _main_body, (zero, zero, zero, zero, zero))

    for r in range(NRING):
        pltpu.make_async_copy(v_hbm.at[:, pl.ds(0, GRAN)], vbufs[r],
                              sems_v[r]).wait()
        pltpu.make_async_copy(w_hbm.at[:, pl.ds(0, GRAN)], wbufs[r],
                              sems_w[r]).wait()

    stage[pl.ds(0, 16)] = s1
    stage[pl.ds(16, 16)] = s2
    stage[pl.ds(32, 16)] = s3
    stage[pl.ds(48, 16)] = t1
    stage[pl.ds(64, 16)] = t2
    pltpu.sync_copy(stage, shared.at[pl.ds(sid * 80, 80)])
    plsc.subcore_barrier()

    @pl.when(sid == 0)
    def _():
        pltpu.sync_copy(shared, red_v)
        a1 = zero
        a2 = zero
        a3 = zero
        a4 = zero
        a5 = zero
        for t in range(NTILES):
            a1 = a1 + red_v[pl.ds(t * 80 + 0, 16)]
            a2 = a2 + red_v[pl.ds(t * 80 + 16, 16)]
            a3 = a3 + red_v[pl.ds(t * 80 + 32, 16)]
            a4 = a4 + red_v[pl.ds(t * 80 + 48, 16)]
            a5 = a5 + red_v[pl.ds(t * 80 + 64, 16)]
        S1 = jnp.sum(a1)
        S2 = jnp.sum(a2)
        S3 = jnp.sum(a3)
        T1 = jnp.sum(a4)
        T2 = jnp.sum(a5)
        res = jnp.where(iota == 0, S1, jnp.zeros((16,), jnp.float32))
        res = jnp.where(iota == 1, S2, res)
        res = jnp.where(iota == 2, S3, res)
        res = jnp.where(iota == 3, T1, res)
        res = jnp.where(iota == 4, T2, res)
        out_stage[...] = res
        pltpu.sync_copy(out_stage,
                        out_hbm.at[pl.ds(pl.multiple_of(cid * 16, 16), 16)])


def kernel(v_table, w_table, v_bias_table, w_bias_table, cooccur,
           feature_idx1, feature_idx2):
    mesh = plsc.VectorSubcoreMesh(
        core_axis_name="c", subcore_axis_name="s", num_cores=NCORES)
    run = pl.kernel(
        _glove_kernel,
        out_type=jax.ShapeDtypeStruct((32,), jnp.float32),
        mesh=mesh,
        compiler_params=pltpu.CompilerParams(
            needs_layout_passes=False,
            use_tc_tiling_on_sc=True,
        ),
        scratch_types=[
            pltpu.VMEM((CHUNK,), jnp.int32),
            pltpu.VMEM((CHUNK,), jnp.int32),
            [pltpu.VMEM((D, GRAN), jnp.float32) for _ in range(NRING)],
            [pltpu.VMEM((D, GRAN), jnp.float32) for _ in range(NRING)],
            pltpu.VMEM((16,), jnp.float32),
            pltpu.VMEM((CHUNK,), jnp.float32),
            pltpu.VMEM((CHUNK,), jnp.float32),
            pltpu.VMEM((CHUNK,), jnp.float32),
            pltpu.VMEM((80,), jnp.float32),
            pltpu.VMEM_SHARED((NTILES * 80,), jnp.float32),
            pltpu.VMEM((NTILES * 80,), jnp.float32),
            pltpu.VMEM((16,), jnp.float32),
            [pltpu.SemaphoreType.DMA for _ in range(NRING)],
            [pltpu.SemaphoreType.DMA for _ in range(NRING)],
            pltpu.SemaphoreType.DMA,
            pltpu.SemaphoreType.DMA,
        ],
    )
    out = run(v_table.T, w_table.T,
              jax.lax.squeeze(v_bias_table, (1,)),
              jax.lax.squeeze(w_bias_table, (1,)),
              cooccur, feature_idx1.astype(jnp.int32),
              feature_idx2.astype(jnp.int32))
    S1 = out[0] + out[16]
    S2 = out[1] + out[17]
    S3 = out[2] + out[18]
    T1 = out[3] + out[19]
    T2 = out[4] + out[20]
    return float(B) * S1 + 2.0 * S2 * T1 + S3 * T2
